# baseline (speedup 1.0000x reference)
import jax
import jax.numpy as jnp
from jax import lax
from jax.experimental import pallas as pl
from jax.experimental.pallas import tpu as pltpu
from jax.experimental.pallas import tpu_sc as plsc

N = 10000
E = 320000
D = 128
DE = 16
H = 4
C = 128
OUT = 24

NC = 2
NS = 16
NW = NC * NS
L = 16
G = 80
EPT = E // NS
EPT_D = E // NW
NP = 10240
NPT = NP // NS
ZR = 16
GD = 80


def _lin_kernel(x_ref, wl_ref, wr_ref, xl_ref, xr_ref):
    xl_ref[0] = jnp.dot(x_ref[...], wl_ref[...], preferred_element_type=jnp.float32)
    xr_ref[0] = jnp.dot(x_ref[...], wr_ref[...], preferred_element_type=jnp.float32)


def _ef_kernel(ea_ref, we_ref, ef_ref):
    ef_ref[0] = jnp.dot(ea_ref[...], we_ref[...], preferred_element_type=jnp.float32)


def _leaky(z):
    return jnp.maximum(z, 0.0) + 0.2 * jnp.minimum(z, 0.0)


_DNUMS = lax.GatherDimensionNumbers(
    offset_dims=(), collapsed_slice_dims=(0,), start_index_map=(0,))


def _bcast_lane(v, t):
    idx = jnp.full((L, 1), t, jnp.int32)
    return lax.gather(v, idx, _DNUMS, (1,),
                      mode=lax.GatherScatterMode.PROMISE_IN_BOUNDS)


def _sc_body(src_hbm, dst_hbm, ea_hbm, xl_hbm, xr_hbm, ef_hbm, att_hbm,
             msg_out, deg_out, den_out, p_stage,
             acc_sh,
             srcbuf, dstbuf, idxbuf, xl_buf, msg_buf, xr_buf, ef_buf,
             att_buf, ea_chunk, pbuf, pb0, pb1,
             sem1, sem2, sem3):
    c = lax.axis_index("c")
    s = lax.axis_index("s")
    tid = c * NS + s

    zero16 = jnp.zeros((L,), jnp.float32)
    lanes = lax.iota(jnp.int32, L)
    e0 = jnp.where(lanes == 0, 1.0, 0.0)
    e1 = jnp.where(lanes == 1, 1.0, 0.0)

    def _zinit(i, _):
        for j in range(C // L):
            ef_buf[i, pl.ds(j * L, L)] = zero16
        return _
    lax.fori_loop(0, ZR, _zinit, None)

    def _rinit(i, _):
        xl_buf[i, pl.ds(DE, L)] = e0
        for j in range(2, C // L):
            xl_buf[i, pl.ds(j * L, L)] = zero16
        return _
    lax.fori_loop(0, GD, _rinit, None)

    zslice = ef_buf.at[pl.ds(0, ZR)]

    for k in range(NPT // ZR):
        pltpu.sync_copy(zslice, acc_sh.at[pl.ds(s * NPT + k * ZR, ZR)])
    plsc.subcore_barrier()

    def _deg_chunk(k, _):
        base = tid * EPT_D + k * GD
        pltpu.sync_copy(dst_hbm.at[pl.ds(base, GD)], dstbuf.at[pl.ds(0, GD)])
        pltpu.sync_copy(ea_hbm.at[pl.ds(base, GD)], ea_chunk)

        def _fill(e, __):
            xl_buf[e, pl.ds(0, DE)] = ea_chunk[e, pl.ds(0, DE)]
            return __
        lax.fori_loop(0, GD, _fill, None)
        pltpu.sync_copy(xl_buf.at[pl.ds(0, GD)],
                        acc_sh.at[dstbuf.at[pl.ds(0, GD)]], add=True)
        return _
    lax.fori_loop(0, EPT_D // GD, _deg_chunk, None)
    plsc.subcore_barrier()
    pltpu.sync_copy(acc_sh.at[pl.ds(s * NPT, NPT)],
                    deg_out.at[c, pl.ds(s * NPT, NPT)])
    for k in range(NPT // ZR):
        pltpu.sync_copy(zslice, acc_sh.at[pl.ds(s * NPT + k * ZR, ZR)])
    plsc.subcore_barrier()

    def _head_body(hi, _hc):
        h = c * 2 + hi
        hN = h * N
        hE = h * E
        pltpu.sync_copy(att_hbm.at[pl.ds(h * C, C)], att_buf)

        def _chunk(k, _):
            base = s * EPT + k * G
            pltpu.sync_copy(src_hbm.at[pl.ds(base, G)], srcbuf)
            pltpu.sync_copy(dst_hbm.at[pl.ds(base, G)], dstbuf)
            for j in range(G // L):
                sl = pl.ds(j * L, L)
                srcbuf[sl] = srcbuf[sl] + hN
                idxbuf[sl] = dstbuf[sl] + hN
            cp1 = pltpu.async_copy(xl_hbm.at[srcbuf], xl_buf, sem1)
            cp2 = pltpu.async_copy(xr_hbm.at[idxbuf], xr_buf, sem2)
            cp3 = pltpu.async_copy(ef_hbm.at[pl.ds(hE + base, G)], ef_buf, sem3)
            cp1.wait()
            cp2.wait()
            cp3.wait()

            def _egroup(g, __):
                row0 = g * L
                phold = zero16
                for t in range(L):
                    row = row0 + t
                    a0 = zero16
                    a1 = zero16
                    xls = []
                    for j in range(C // L):
                        sl = pl.ds(j * L, L)
                        xlv = xl_buf[row, sl]
                        xls.append(xlv)
                        z = xlv + xr_buf[row, sl] + ef_buf[row, sl]
                        if j & 1:
                            a1 = a1 + _leaky(z) * att_buf[sl]
                        else:
                            a0 = a0 + _leaky(z) * att_buf[sl]
                    a_acc = a0 + a1
                    for sh in (8, 4, 2, 1):
                        shuf = lax.gather(
                            a_acc, (lanes ^ sh)[:, None], _DNUMS, (1,),
                            mode=lax.GatherScatterMode.PROMISE_IN_BOUNDS)
                        a_acc = a_acc + shuf
                    pv = jnp.exp(a_acc)
                    for j in range(C // L):
                        msg_buf[row, pl.ds(j * L, L)] = pv * xls[j]
                    oht = jnp.where(lanes == t, 1.0, 0.0)
                    phold = phold + pv * oht
                pbuf[pl.ds(row0, L)] = phold
                return __
            lax.fori_loop(0, G // L, _egroup, None)
            pltpu.sync_copy(msg_buf, acc_sh.at[dstbuf], add=True)
            pltpu.sync_copy(pbuf, p_stage.at[h, pl.ds(base, G)])
            return _
        lax.fori_loop(0, EPT // G, _chunk, None)

        plsc.subcore_barrier()
        pltpu.sync_copy(acc_sh.at[pl.ds(s * NPT, NPT)],
                        msg_out.at[h, pl.ds(s * NPT, NPT)])
        lax.fori_loop(0, ZR, _zinit, None)
        for k in range(NPT // ZR):
            pltpu.sync_copy(zslice, acc_sh.at[pl.ds(s * NPT + k * ZR, ZR)])
        plsc.subcore_barrier()
        return _hc
    lax.fori_loop(0, 2, _head_body, None)

    def _dinit(i, _):
        for j in range(1, C // L):
            xl_buf[i, pl.ds(j * L, L)] = zero16
        return _
    lax.fori_loop(0, G, _dinit, None)

    h0 = c * 2
    h1 = c * 2 + 1

    def _den_chunk(k, _):
        base = s * EPT + k * G
        pltpu.sync_copy(dst_hbm.at[pl.ds(base, G)], dstbuf)
        pltpu.sync_copy(p_stage.at[h0, pl.ds(base, G)], pb0)
        pltpu.sync_copy(p_stage.at[h1, pl.ds(base, G)], pb1)

        def _dgroup(g, __):
            row0 = g * L
            pv0 = pb0[pl.ds(row0, L)]
            pv1 = pb1[pl.ds(row0, L)]
            for t in range(L):
                p0t = _bcast_lane(pv0, t)
                p1t = _bcast_lane(pv1, t)
                xl_buf[row0 + t, pl.ds(0, L)] = p0t * e0 + p1t * e1
            return __
        lax.fori_loop(0, G // L, _dgroup, None)
        pltpu.sync_copy(xl_buf, acc_sh.at[dstbuf], add=True)
        return _
    lax.fori_loop(0, EPT // G, _den_chunk, None)
    plsc.subcore_barrier()
    pltpu.sync_copy(acc_sh.at[pl.ds(s * NPT, NPT)],
                    den_out.at[c, pl.ds(s * NPT, NPT)])


def _sc_call(src, dst, edge_attr, xl2, xr2, ef2, att_flat):
    mesh = plsc.VectorSubcoreMesh(core_axis_name="c", subcore_axis_name="s")
    f = pl.kernel(
        _sc_body,
        out_type=[
            jax.ShapeDtypeStruct((H, NP, C), jnp.float32),
            jax.ShapeDtypeStruct((NC, NP, C), jnp.float32),
            jax.ShapeDtypeStruct((NC, NP, C), jnp.float32),
            jax.ShapeDtypeStruct((H, E), jnp.float32),
        ],
        mesh=mesh,
        compiler_params=pltpu.CompilerParams(use_tc_tiling_on_sc=False),
        scratch_types=[
            pltpu.VMEM_SHARED((NP, C), jnp.float32),
            pltpu.VMEM((G,), jnp.int32),
            pltpu.VMEM((G,), jnp.int32),
            pltpu.VMEM((G,), jnp.int32),
            pltpu.VMEM((G, C), jnp.float32),
            pltpu.VMEM((G, C), jnp.float32),
            pltpu.VMEM((G, C), jnp.float32),
            pltpu.VMEM((G, C), jnp.float32),
            pltpu.VMEM((C,), jnp.float32),
            pltpu.VMEM((GD, DE), jnp.float32),
            pltpu.VMEM((G,), jnp.float32),
            pltpu.VMEM((G,), jnp.float32),
            pltpu.VMEM((G,), jnp.float32),
            pltpu.SemaphoreType.DMA,
            pltpu.SemaphoreType.DMA,
            pltpu.SemaphoreType.DMA,
        ],
    )
    return f(src, dst, edge_attr, xl2, xr2, ef2, att_flat)


def _epi_kernel(msg_ref, xl_ref, xr_ref, deg_ref, den_ref, we_ref, att_ref,
                bias_ref, wp_ref, bp_ref, out_ref):
    ea_sum = deg_ref[0, :, 0:DE] + deg_ref[1, :, 0:DE]
    deg = deg_ref[0, :, DE:DE + 1] + deg_ref[1, :, DE:DE + 1]
    ea_mean = ea_sum / jnp.maximum(deg, 1.0)
    ef_self = jnp.dot(ea_mean, we_ref[...], preferred_element_type=jnp.float32)
    acc = jnp.zeros((out_ref.shape[0], OUT), jnp.float32)
    for h in range(H):
        xlh = xl_ref[h]
        z = xlh + xr_ref[h] + ef_self[:, h * C:(h + 1) * C]
        z = _leaky(z)
        a = jnp.sum(z * att_ref[h][None, :], axis=1, keepdims=True)
        p = jnp.exp(a)
        msg = msg_ref[h] + p * xlh
        den = den_ref[h // 2, :, (h % 2):(h % 2) + 1] + p + 1e-16
        hh = msg / den + bias_ref[h][None, :]
        acc = acc + jnp.dot(hh, wp_ref[h], preferred_element_type=jnp.float32)
    out_ref[...] = acc + bp_ref[...][None, :]


def kernel(x, edge_index, edge_attr, W_l, W_r, W_e, att, bias, W_pred, b_pred):
    src = edge_index[0].astype(jnp.int32)
    dst = edge_index[1].astype(jnp.int32)

    nb = 10
    NB = N // nb
    xl, xr = pl.pallas_call(
        _lin_kernel,
        grid=(H, nb),
        in_specs=[
            pl.BlockSpec((NB, D), lambda h, i: (i, 0)),
            pl.BlockSpec((D, C), lambda h, i: (0, h)),
            pl.BlockSpec((D, C), lambda h, i: (0, h)),
        ],
        out_specs=[
            pl.BlockSpec((1, NB, C), lambda h, i: (h, i, 0)),
            pl.BlockSpec((1, NB, C), lambda h, i: (h, i, 0)),
        ],
        out_shape=[
            jax.ShapeDtypeStruct((H, N, C), jnp.float32),
            jax.ShapeDtypeStruct((H, N, C), jnp.float32),
        ],
    )(x, W_l, W_r)

    eb = 40
    ef = pl.pallas_call(
        _ef_kernel,
        grid=(H, eb),
        in_specs=[
            pl.BlockSpec((E // eb, DE), lambda h, i: (i, 0)),
            pl.BlockSpec((DE, C), lambda h, i: (0, h)),
        ],
        out_specs=pl.BlockSpec((1, E // eb, C), lambda h, i: (h, i, 0)),
        out_shape=jax.ShapeDtypeStruct((H, E, C), jnp.float32),
    )(edge_attr, W_e)

    xl2 = xl.reshape(H * N, C)
    xr2 = xr.reshape(H * N, C)
    ef2 = ef.reshape(H * E, C)
    att_flat = att.reshape(H * C)

    msg_out, deg_out, den_out, _p = _sc_call(src, dst, edge_attr, xl2, xr2,
                                             ef2, att_flat)

    bias_h = bias.reshape(H, C)
    wp_h = W_pred.reshape(H, C, OUT)
    out = pl.pallas_call(
        _epi_kernel,
        grid=(nb,),
        in_specs=[
            pl.BlockSpec((H, NB, C), lambda i: (0, i, 0)),
            pl.BlockSpec((H, NB, C), lambda i: (0, i, 0)),
            pl.BlockSpec((H, NB, C), lambda i: (0, i, 0)),
            pl.BlockSpec((NC, NB, C), lambda i: (0, i, 0)),
            pl.BlockSpec((NC, NB, C), lambda i: (0, i, 0)),
            pl.BlockSpec((DE, H * C), lambda i: (0, 0)),
            pl.BlockSpec((H, C), lambda i: (0, 0)),
            pl.BlockSpec((H, C), lambda i: (0, 0)),
            pl.BlockSpec((H, C, OUT), lambda i: (0, 0, 0)),
            pl.BlockSpec((OUT,), lambda i: (0,)),
        ],
        out_specs=pl.BlockSpec((NB, OUT), lambda i: (i, 0)),
        out_shape=jax.ShapeDtypeStruct((N, OUT), jnp.float32),
    )(msg_out, xl, xr, deg_out, den_out, W_e, att, bias_h, wp_h, b_pred)
    return out


# speedup vs baseline: 1.0962x; 1.0962x over previous
import jax
import jax.numpy as jnp
from jax import lax
from jax.experimental import pallas as pl
from jax.experimental.pallas import tpu as pltpu
from jax.experimental.pallas import tpu_sc as plsc

N = 10000
E = 320000
D = 128
DE = 16
H = 4
C = 128
OUT = 24

NC = 2
NS = 16
NW = NC * NS
L = 16
G = 80
EPT = E // NS
EPT_D = E // NW
NP = 10240
NPT = NP // NS
ZR = 16
GD = 80
SUP = 10


def _lin_kernel(x_ref, wl_ref, wr_ref, xl_ref, xr_ref):
    xl_ref[0] = jnp.dot(x_ref[...], wl_ref[...], preferred_element_type=jnp.float32)
    xr_ref[0] = jnp.dot(x_ref[...], wr_ref[...], preferred_element_type=jnp.float32)


def _ef_kernel(ea_ref, we_ref, ef_ref):
    ef_ref[0] = jnp.dot(ea_ref[...], we_ref[...], preferred_element_type=jnp.float32)


def _leaky(z):
    return jnp.maximum(z, 0.0) + 0.2 * jnp.minimum(z, 0.0)


_DNUMS = lax.GatherDimensionNumbers(
    offset_dims=(), collapsed_slice_dims=(0,), start_index_map=(0,))


def _bcast_lane(v, t):
    idx = jnp.full((L, 1), t, jnp.int32)
    return lax.gather(v, idx, _DNUMS, (1,),
                      mode=lax.GatherScatterMode.PROMISE_IN_BOUNDS)


def _sc_body(src2d_hbm, dst2d_hbm, ea_hbm, xl_hbm, xr_hbm,
             ef_hbm, att_hbm,
             msg_out, deg_out, den_out, p_stage,
             acc_sh,
             srcbig, dstbig, idxbuf, xl_buf, msg_buf, xr_buf, ef_buf,
             att_buf, ea_chunk, pbig, pb0, pb1, dstbuf,
             sem1, sem2, sem3):
    c = lax.axis_index("c")
    s = lax.axis_index("s")
    tid = c * NS + s

    zero16 = jnp.zeros((L,), jnp.float32)
    lanes = lax.iota(jnp.int32, L)
    e0 = jnp.where(lanes == 0, 1.0, 0.0)
    e1 = jnp.where(lanes == 1, 1.0, 0.0)

    def _zinit(i, _):
        for j in range(C // L):
            ef_buf[i, pl.ds(j * L, L)] = zero16
        return _
    lax.fori_loop(0, ZR, _zinit, None)

    def _rinit(i, _):
        xl_buf[i, pl.ds(DE, L)] = e0
        for j in range(2, C // L):
            xl_buf[i, pl.ds(j * L, L)] = zero16
        return _
    lax.fori_loop(0, GD, _rinit, None)

    zslice = ef_buf.at[pl.ds(0, ZR)]

    for k in range(NPT // ZR):
        pltpu.sync_copy(zslice, acc_sh.at[pl.ds(s * NPT + k * ZR, ZR)])
    plsc.subcore_barrier()

    def _deg_chunk(k, _):
        base = tid * EPT_D + k * GD
        pltpu.sync_copy(dst2d_hbm.at[tid * (EPT_D // GD) + k],
                        dstbuf.at[pl.ds(0, GD)])
        pltpu.sync_copy(ea_hbm.at[pl.ds(base, GD)], ea_chunk)

        def _fill(e, __):
            xl_buf[e, pl.ds(0, DE)] = ea_chunk[e, pl.ds(0, DE)]
            return __
        lax.fori_loop(0, GD, _fill, None)
        pltpu.sync_copy(xl_buf.at[pl.ds(0, GD)],
                        acc_sh.at[dstbuf.at[pl.ds(0, GD)]], add=True)
        return _
    lax.fori_loop(0, EPT_D // GD, _deg_chunk, None)
    plsc.subcore_barrier()
    pltpu.sync_copy(acc_sh.at[pl.ds(s * NPT, NPT)],
                    deg_out.at[c, pl.ds(s * NPT, NPT)])
    for k in range(NPT // ZR):
        pltpu.sync_copy(zslice, acc_sh.at[pl.ds(s * NPT + k * ZR, ZR)])
    plsc.subcore_barrier()

    def _head_body(hi, _hc):
        h = c * 2 + hi
        hN = h * N
        hE = h * E
        pltpu.sync_copy(att_hbm.at[pl.ds(h * C, C)], att_buf)

        def _sup(u, _):
            rowbase = s * (EPT // G) + u * SUP
            pltpu.sync_copy(src2d_hbm.at[pl.ds(rowbase, SUP)], srcbig)
            pltpu.sync_copy(dst2d_hbm.at[pl.ds(rowbase, SUP)], dstbig)

            def _off(r, __):
                for j in range(G // L):
                    sl = pl.ds(j * L, L)
                    srcbig[r, sl] = srcbig[r, sl] + hN
                return __
            lax.fori_loop(0, SUP, _off, None)

            def _chunk(kk, __):
                base = (rowbase + kk) * G
                for j in range(G // L):
                    sl = pl.ds(j * L, L)
                    idxbuf[sl] = dstbig[kk, sl] + hN
                cp1 = pltpu.async_copy(xl_hbm.at[srcbig.at[kk]], xl_buf, sem1)
                cp2 = pltpu.async_copy(xr_hbm.at[idxbuf], xr_buf, sem2)
                cp3 = pltpu.async_copy(ef_hbm.at[pl.ds(hE + base, G)], ef_buf,
                                       sem3)
                cp1.wait()
                cp2.wait()
                cp3.wait()

                def _egroup(g, ___):
                    row0 = g * L
                    phold = zero16
                    for t in range(L):
                        row = row0 + t
                        a0 = zero16
                        a1 = zero16
                        xls = []
                        for j in range(C // L):
                            sl = pl.ds(j * L, L)
                            xlv = xl_buf[row, sl]
                            xls.append(xlv)
                            z = xlv + xr_buf[row, sl] + ef_buf[row, sl]
                            if j & 1:
                                a1 = a1 + _leaky(z) * att_buf[sl]
                            else:
                                a0 = a0 + _leaky(z) * att_buf[sl]
                        a_acc = a0 + a1
                        for sh in (8, 4, 2, 1):
                            shuf = lax.gather(
                                a_acc, (lanes ^ sh)[:, None], _DNUMS, (1,),
                                mode=lax.GatherScatterMode.PROMISE_IN_BOUNDS)
                            a_acc = a_acc + shuf
                        pv = jnp.exp(a_acc)
                        for j in range(C // L):
                            msg_buf[row, pl.ds(j * L, L)] = pv * xls[j]
                        oht = jnp.where(lanes == t, 1.0, 0.0)
                        phold = phold + pv * oht
                    pbig[kk, pl.ds(row0, L)] = phold
                    return ___
                lax.fori_loop(0, G // L, _egroup, None)
                pltpu.sync_copy(msg_buf, acc_sh.at[dstbig.at[kk]], add=True)
                return __
            lax.fori_loop(0, SUP, _chunk, None)
            pltpu.sync_copy(pbig, p_stage.at[h, pl.ds(rowbase, SUP)])
            return _
        lax.fori_loop(0, EPT // G // SUP, _sup, None)

        plsc.subcore_barrier()
        pltpu.sync_copy(acc_sh.at[pl.ds(s * NPT, NPT)],
                        msg_out.at[h, pl.ds(s * NPT, NPT)])
        lax.fori_loop(0, ZR, _zinit, None)
        for k in range(NPT // ZR):
            pltpu.sync_copy(zslice, acc_sh.at[pl.ds(s * NPT + k * ZR, ZR)])
        plsc.subcore_barrier()
        return _hc
    lax.fori_loop(0, 2, _head_body, None)

    def _dinit(i, _):
        for j in range(1, C // L):
            xl_buf[i, pl.ds(j * L, L)] = zero16
        return _
    lax.fori_loop(0, G, _dinit, None)

    h0 = c * 2
    h1 = c * 2 + 1

    def _den_chunk(k, _):
        base = s * EPT + k * G
        rowidx = s * (EPT // G) + k
        pltpu.sync_copy(dst2d_hbm.at[rowidx], dstbuf)
        pltpu.sync_copy(p_stage.at[h0, rowidx], pb0)
        pltpu.sync_copy(p_stage.at[h1, rowidx], pb1)

        def _dgroup(g, __):
            row0 = g * L
            pv0 = pb0[pl.ds(row0, L)]
            pv1 = pb1[pl.ds(row0, L)]
            for t in range(L):
                p0t = _bcast_lane(pv0, t)
                p1t = _bcast_lane(pv1, t)
                xl_buf[row0 + t, pl.ds(0, L)] = p0t * e0 + p1t * e1
            return __
        lax.fori_loop(0, G // L, _dgroup, None)
        pltpu.sync_copy(xl_buf, acc_sh.at[dstbuf], add=True)
        return _
    lax.fori_loop(0, EPT // G, _den_chunk, None)
    plsc.subcore_barrier()
    pltpu.sync_copy(acc_sh.at[pl.ds(s * NPT, NPT)],
                    den_out.at[c, pl.ds(s * NPT, NPT)])


def _sc_call(src, dst, edge_attr, xl2, xr2, ef2, att_flat):
    src2d = src.reshape(E // G, G)
    dst2d = dst.reshape(E // G, G)
    mesh = plsc.VectorSubcoreMesh(core_axis_name="c", subcore_axis_name="s")
    f = pl.kernel(
        _sc_body,
        out_type=[
            jax.ShapeDtypeStruct((H, NP, C), jnp.float32),
            jax.ShapeDtypeStruct((NC, NP, C), jnp.float32),
            jax.ShapeDtypeStruct((NC, NP, C), jnp.float32),
            jax.ShapeDtypeStruct((H, E // G, G), jnp.float32),
        ],
        mesh=mesh,
        compiler_params=pltpu.CompilerParams(use_tc_tiling_on_sc=False),
        scratch_types=[
            pltpu.VMEM_SHARED((NP, C), jnp.float32),
            pltpu.VMEM((SUP, G), jnp.int32),
            pltpu.VMEM((SUP, G), jnp.int32),
            pltpu.VMEM((G,), jnp.int32),
            pltpu.VMEM((G, C), jnp.float32),
            pltpu.VMEM((G, C), jnp.float32),
            pltpu.VMEM((G, C), jnp.float32),
            pltpu.VMEM((G, C), jnp.float32),
            pltpu.VMEM((C,), jnp.float32),
            pltpu.VMEM((GD, DE), jnp.float32),
            pltpu.VMEM((SUP, G), jnp.float32),
            pltpu.VMEM((G,), jnp.float32),
            pltpu.VMEM((G,), jnp.float32),
            pltpu.VMEM((G,), jnp.int32),
            pltpu.SemaphoreType.DMA,
            pltpu.SemaphoreType.DMA,
            pltpu.SemaphoreType.DMA,
        ],
    )
    return f(src2d, dst2d, edge_attr, xl2, xr2, ef2, att_flat)


def _epi_kernel(msg_ref, xl_ref, xr_ref, deg_ref, den_ref, we_ref, att_ref,
                bias_ref, wp_ref, bp_ref, out_ref):
    ea_sum = deg_ref[0, :, 0:DE] + deg_ref[1, :, 0:DE]
    deg = deg_ref[0, :, DE:DE + 1] + deg_ref[1, :, DE:DE + 1]
    ea_mean = ea_sum / jnp.maximum(deg, 1.0)
    ef_self = jnp.dot(ea_mean, we_ref[...], preferred_element_type=jnp.float32)
    acc = jnp.zeros((out_ref.shape[0], OUT), jnp.float32)
    for h in range(H):
        xlh = xl_ref[h]
        z = xlh + xr_ref[h] + ef_self[:, h * C:(h + 1) * C]
        z = _leaky(z)
        a = jnp.sum(z * att_ref[h][None, :], axis=1, keepdims=True)
        p = jnp.exp(a)
        msg = msg_ref[h] + p * xlh
        den = den_ref[h // 2, :, (h % 2):(h % 2) + 1] + p + 1e-16
        hh = msg / den + bias_ref[h][None, :]
        acc = acc + jnp.dot(hh, wp_ref[h], preferred_element_type=jnp.float32)
    out_ref[...] = acc + bp_ref[...][None, :]


def kernel(x, edge_index, edge_attr, W_l, W_r, W_e, att, bias, W_pred, b_pred):
    src = edge_index[0].astype(jnp.int32)
    dst = edge_index[1].astype(jnp.int32)

    nb = 10
    NB = N // nb
    xl, xr = pl.pallas_call(
        _lin_kernel,
        grid=(H, nb),
        in_specs=[
            pl.BlockSpec((NB, D), lambda h, i: (i, 0)),
            pl.BlockSpec((D, C), lambda h, i: (0, h)),
            pl.BlockSpec((D, C), lambda h, i: (0, h)),
        ],
        out_specs=[
            pl.BlockSpec((1, NB, C), lambda h, i: (h, i, 0)),
            pl.BlockSpec((1, NB, C), lambda h, i: (h, i, 0)),
        ],
        out_shape=[
            jax.ShapeDtypeStruct((H, N, C), jnp.float32),
            jax.ShapeDtypeStruct((H, N, C), jnp.float32),
        ],
    )(x, W_l, W_r)

    eb = 40
    ef = pl.pallas_call(
        _ef_kernel,
        grid=(H, eb),
        in_specs=[
            pl.BlockSpec((E // eb, DE), lambda h, i: (i, 0)),
            pl.BlockSpec((DE, C), lambda h, i: (0, h)),
        ],
        out_specs=pl.BlockSpec((1, E // eb, C), lambda h, i: (h, i, 0)),
        out_shape=jax.ShapeDtypeStruct((H, E, C), jnp.float32),
    )(edge_attr, W_e)

    xl2 = xl.reshape(H * N, C)
    xr2 = xr.reshape(H * N, C)
    ef2 = ef.reshape(H * E, C)
    att_flat = att.reshape(H * C)

    msg_out, deg_out, den_out, _p = _sc_call(src, dst, edge_attr, xl2, xr2,
                                             ef2, att_flat)

    bias_h = bias.reshape(H, C)
    wp_h = W_pred.reshape(H, C, OUT)
    out = pl.pallas_call(
        _epi_kernel,
        grid=(nb,),
        in_specs=[
            pl.BlockSpec((H, NB, C), lambda i: (0, i, 0)),
            pl.BlockSpec((H, NB, C), lambda i: (0, i, 0)),
            pl.BlockSpec((H, NB, C), lambda i: (0, i, 0)),
            pl.BlockSpec((NC, NB, C), lambda i: (0, i, 0)),
            pl.BlockSpec((NC, NB, C), lambda i: (0, i, 0)),
            pl.BlockSpec((DE, H * C), lambda i: (0, 0)),
            pl.BlockSpec((H, C), lambda i: (0, 0)),
            pl.BlockSpec((H, C), lambda i: (0, 0)),
            pl.BlockSpec((H, C, OUT), lambda i: (0, 0, 0)),
            pl.BlockSpec((OUT,), lambda i: (0,)),
        ],
        out_specs=pl.BlockSpec((NB, OUT), lambda i: (i, 0)),
        out_shape=jax.ShapeDtypeStruct((N, OUT), jnp.float32),
    )(msg_out, xl, xr, deg_out, den_out, W_e, att, bias_h, wp_h, b_pred)
    return out


# speedup vs baseline: 1.1668x; 1.0644x over previous
import jax
import jax.numpy as jnp
from jax import lax
from jax.experimental import pallas as pl
from jax.experimental.pallas import tpu as pltpu
from jax.experimental.pallas import tpu_sc as plsc

N = 10000
E = 320000
D = 128
DE = 16
H = 4
C = 128
OUT = 24

NC = 2
NS = 16
NW = NC * NS
L = 16
G = 80
EPT = E // NS
EPT_D = E // NW
NP = 10240
NPT = NP // NS
ZR = 16
GD = 80
SUP = 10


def _lin_kernel(x_ref, wl_ref, wr_ref, xl_ref, xr_ref):
    xl_ref[0] = jnp.dot(x_ref[...], wl_ref[...], preferred_element_type=jnp.float32)
    xr_ref[0] = jnp.dot(x_ref[...], wr_ref[...], preferred_element_type=jnp.float32)


def _ef_kernel(ea_ref, we_ref, ef_ref):
    ef_ref[0] = jnp.dot(ea_ref[...], we_ref[...], preferred_element_type=jnp.float32)


def _leaky(z):
    return jnp.maximum(z, 0.0) + 0.2 * jnp.minimum(z, 0.0)


_DNUMS = lax.GatherDimensionNumbers(
    offset_dims=(), collapsed_slice_dims=(0,), start_index_map=(0,))


def _bcast_lane(v, t):
    idx = jnp.full((L, 1), t, jnp.int32)
    return lax.gather(v, idx, _DNUMS, (1,),
                      mode=lax.GatherScatterMode.PROMISE_IN_BOUNDS)


def _sc_body(src2d_hbm, dst2d_hbm, ea_hbm, xl_hbm, xr_hbm,
             ef_hbm, att_hbm,
             msg_out, deg_out, den_out, p_stage,
             acc_sh,
             srcbig, dstbig, idxbuf, xl_buf, msg_buf, xr_buf, ef_buf,
             att_buf, ea_chunk, pbig, pb1big, dstbuf,
             sem1, sem2, sem3):
    c = lax.axis_index("c")
    s = lax.axis_index("s")
    tid = c * NS + s

    zero16 = jnp.zeros((L,), jnp.float32)
    lanes = lax.iota(jnp.int32, L)
    e0 = jnp.where(lanes == 0, 1.0, 0.0)
    e1 = jnp.where(lanes == 1, 1.0, 0.0)

    def _zinit(i, _):
        for j in range(C // L):
            ef_buf[i, pl.ds(j * L, L)] = zero16
        return _
    lax.fori_loop(0, ZR, _zinit, None)

    def _rinit(i, _):
        xl_buf[i, pl.ds(DE, L)] = e0
        for j in range(2, C // L):
            xl_buf[i, pl.ds(j * L, L)] = zero16
        return _
    lax.fori_loop(0, GD, _rinit, None)

    zslice = ef_buf.at[pl.ds(0, ZR)]

    for k in range(NPT // ZR):
        pltpu.sync_copy(zslice, acc_sh.at[pl.ds(s * NPT + k * ZR, ZR)])
    plsc.subcore_barrier()

    def _deg_chunk(k, _):
        base = tid * EPT_D + k * GD
        pltpu.sync_copy(dst2d_hbm.at[tid * (EPT_D // GD) + k],
                        dstbuf.at[pl.ds(0, GD)])
        pltpu.sync_copy(ea_hbm.at[pl.ds(base, GD)], ea_chunk)

        def _fill(e, __):
            xl_buf[e, pl.ds(0, DE)] = ea_chunk[e, pl.ds(0, DE)]
            return __
        lax.fori_loop(0, GD, _fill, None)
        pltpu.sync_copy(xl_buf.at[pl.ds(0, GD)],
                        acc_sh.at[dstbuf.at[pl.ds(0, GD)]], add=True)
        return _
    lax.fori_loop(0, EPT_D // GD, _deg_chunk, None)
    plsc.subcore_barrier()
    pltpu.sync_copy(acc_sh.at[pl.ds(s * NPT, NPT)],
                    deg_out.at[c, pl.ds(s * NPT, NPT)])
    for k in range(NPT // ZR):
        pltpu.sync_copy(zslice, acc_sh.at[pl.ds(s * NPT + k * ZR, ZR)])
    plsc.subcore_barrier()

    def _head_body(hi, _hc):
        h = c * 2 + hi
        hN = h * N
        hE = h * E
        pltpu.sync_copy(att_hbm.at[pl.ds(h * C, C)], att_buf)

        def _sup(u, _):
            rowbase = s * (EPT // G) + u * SUP
            pltpu.sync_copy(src2d_hbm.at[pl.ds(rowbase, SUP)], srcbig)
            pltpu.sync_copy(dst2d_hbm.at[pl.ds(rowbase, SUP)], dstbig)

            def _off(r, __):
                for j in range(G // L):
                    sl = pl.ds(j * L, L)
                    srcbig[r, sl] = srcbig[r, sl] + hN
                return __
            lax.fori_loop(0, SUP, _off, None)

            def _chunk(kk, __):
                base = (rowbase + kk) * G
                for j in range(G // L):
                    sl = pl.ds(j * L, L)
                    idxbuf[sl] = dstbig[kk, sl] + hN
                cp1 = pltpu.async_copy(xl_hbm.at[srcbig.at[kk]], xl_buf, sem1)
                cp2 = pltpu.async_copy(xr_hbm.at[idxbuf], xr_buf, sem2)
                cp3 = pltpu.async_copy(ef_hbm.at[pl.ds(hE + base, G)], ef_buf,
                                       sem3)
                cp1.wait()
                cp2.wait()
                cp3.wait()

                def _egroup(g, ___):
                    row0 = g * L
                    phold = zero16
                    for t in range(L):
                        row = row0 + t
                        a0 = zero16
                        a1 = zero16
                        xls = []
                        for j in range(C // L):
                            sl = pl.ds(j * L, L)
                            xlv = xl_buf[row, sl]
                            xls.append(xlv)
                            z = xlv + xr_buf[row, sl] + ef_buf[row, sl]
                            if j & 1:
                                a1 = a1 + _leaky(z) * att_buf[sl]
                            else:
                                a0 = a0 + _leaky(z) * att_buf[sl]
                        a_acc = a0 + a1
                        for sh in (8, 4, 2, 1):
                            shuf = lax.gather(
                                a_acc, (lanes ^ sh)[:, None], _DNUMS, (1,),
                                mode=lax.GatherScatterMode.PROMISE_IN_BOUNDS)
                            a_acc = a_acc + shuf
                        pv = jnp.exp(a_acc)
                        for j in range(C // L):
                            msg_buf[row, pl.ds(j * L, L)] = pv * xls[j]
                        oht = jnp.where(lanes == t, 1.0, 0.0)
                        phold = phold + pv * oht
                    pbig[kk, pl.ds(row0, L)] = phold
                    return ___
                lax.fori_loop(0, G // L, _egroup, None)
                pltpu.sync_copy(msg_buf, acc_sh.at[dstbig.at[kk]], add=True)
                return __
            lax.fori_loop(0, SUP, _chunk, None)
            pltpu.sync_copy(pbig, p_stage.at[h, pl.ds(rowbase, SUP)])
            return _
        lax.fori_loop(0, EPT // G // SUP, _sup, None)

        plsc.subcore_barrier()
        pltpu.sync_copy(acc_sh.at[pl.ds(s * NPT, NPT)],
                        msg_out.at[h, pl.ds(s * NPT, NPT)])
        lax.fori_loop(0, ZR, _zinit, None)
        for k in range(NPT // ZR):
            pltpu.sync_copy(zslice, acc_sh.at[pl.ds(s * NPT + k * ZR, ZR)])
        plsc.subcore_barrier()
        return _hc
    lax.fori_loop(0, 2, _head_body, None)

    def _dinit(i, _):
        for j in range(1, C // L):
            xl_buf[i, pl.ds(j * L, L)] = zero16
        return _
    lax.fori_loop(0, G, _dinit, None)

    h0 = c * 2
    h1 = c * 2 + 1

    def _den_sup(u, _):
        rowbase = s * (EPT // G) + u * SUP
        pltpu.sync_copy(dst2d_hbm.at[pl.ds(rowbase, SUP)], dstbig)
        pltpu.sync_copy(p_stage.at[h0, pl.ds(rowbase, SUP)], pbig)
        pltpu.sync_copy(p_stage.at[h1, pl.ds(rowbase, SUP)], pb1big)

        def _den_chunk(kk, __):
            def _dgroup(g, ___):
                row0 = g * L
                pv0 = pbig[kk, pl.ds(row0, L)]
                pv1 = pb1big[kk, pl.ds(row0, L)]
                for t in range(L):
                    p0t = _bcast_lane(pv0, t)
                    p1t = _bcast_lane(pv1, t)
                    xl_buf[row0 + t, pl.ds(0, L)] = p0t * e0 + p1t * e1
                return ___
            lax.fori_loop(0, G // L, _dgroup, None)
            pltpu.sync_copy(xl_buf, acc_sh.at[dstbig.at[kk]], add=True)
            return __
        lax.fori_loop(0, SUP, _den_chunk, None)
        return _
    lax.fori_loop(0, EPT // G // SUP, _den_sup, None)
    plsc.subcore_barrier()
    pltpu.sync_copy(acc_sh.at[pl.ds(s * NPT, NPT)],
                    den_out.at[c, pl.ds(s * NPT, NPT)])


def _sc_call(src, dst, edge_attr, xl2, xr2, ef2, att_flat):
    src2d = src.reshape(E // G, G)
    dst2d = dst.reshape(E // G, G)
    mesh = plsc.VectorSubcoreMesh(core_axis_name="c", subcore_axis_name="s")
    f = pl.kernel(
        _sc_body,
        out_type=[
            jax.ShapeDtypeStruct((H, NP, C), jnp.float32),
            jax.ShapeDtypeStruct((NC, NP, C), jnp.float32),
            jax.ShapeDtypeStruct((NC, NP, C), jnp.float32),
            jax.ShapeDtypeStruct((H, E // G, G), jnp.float32),
        ],
        mesh=mesh,
        compiler_params=pltpu.CompilerParams(use_tc_tiling_on_sc=False),
        scratch_types=[
            pltpu.VMEM_SHARED((NP, C), jnp.float32),
            pltpu.VMEM((SUP, G), jnp.int32),
            pltpu.VMEM((SUP, G), jnp.int32),
            pltpu.VMEM((G,), jnp.int32),
            pltpu.VMEM((G, C), jnp.float32),
            pltpu.VMEM((G, C), jnp.float32),
            pltpu.VMEM((G, C), jnp.float32),
            pltpu.VMEM((G, C), jnp.float32),
            pltpu.VMEM((C,), jnp.float32),
            pltpu.VMEM((GD, DE), jnp.float32),
            pltpu.VMEM((SUP, G), jnp.float32),
            pltpu.VMEM((SUP, G), jnp.float32),
            pltpu.VMEM((G,), jnp.int32),
            pltpu.SemaphoreType.DMA,
            pltpu.SemaphoreType.DMA,
            pltpu.SemaphoreType.DMA,
        ],
    )
    return f(src2d, dst2d, edge_attr, xl2, xr2, ef2, att_flat)


def _epi_kernel(msg_ref, xl_ref, xr_ref, deg_ref, den_ref, we_ref, att_ref,
                bias_ref, wp_ref, bp_ref, out_ref):
    ea_sum = deg_ref[0, :, 0:DE] + deg_ref[1, :, 0:DE]
    deg = deg_ref[0, :, DE:DE + 1] + deg_ref[1, :, DE:DE + 1]
    ea_mean = ea_sum / jnp.maximum(deg, 1.0)
    ef_self = jnp.dot(ea_mean, we_ref[...], preferred_element_type=jnp.float32)
    acc = jnp.zeros((out_ref.shape[0], OUT), jnp.float32)
    for h in range(H):
        xlh = xl_ref[h]
        z = xlh + xr_ref[h] + ef_self[:, h * C:(h + 1) * C]
        z = _leaky(z)
        a = jnp.sum(z * att_ref[h][None, :], axis=1, keepdims=True)
        p = jnp.exp(a)
        msg = msg_ref[h] + p * xlh
        den = den_ref[h // 2, :, (h % 2):(h % 2) + 1] + p + 1e-16
        hh = msg / den + bias_ref[h][None, :]
        acc = acc + jnp.dot(hh, wp_ref[h], preferred_element_type=jnp.float32)
    out_ref[...] = acc + bp_ref[...][None, :]


def kernel(x, edge_index, edge_attr, W_l, W_r, W_e, att, bias, W_pred, b_pred):
    src = edge_index[0].astype(jnp.int32)
    dst = edge_index[1].astype(jnp.int32)

    nb = 10
    NB = N // nb
    xl, xr = pl.pallas_call(
        _lin_kernel,
        grid=(H, nb),
        in_specs=[
            pl.BlockSpec((NB, D), lambda h, i: (i, 0)),
            pl.BlockSpec((D, C), lambda h, i: (0, h)),
            pl.BlockSpec((D, C), lambda h, i: (0, h)),
        ],
        out_specs=[
            pl.BlockSpec((1, NB, C), lambda h, i: (h, i, 0)),
            pl.BlockSpec((1, NB, C), lambda h, i: (h, i, 0)),
        ],
        out_shape=[
            jax.ShapeDtypeStruct((H, N, C), jnp.float32),
            jax.ShapeDtypeStruct((H, N, C), jnp.float32),
        ],
    )(x, W_l, W_r)

    eb = 40
    ef = pl.pallas_call(
        _ef_kernel,
        grid=(H, eb),
        in_specs=[
            pl.BlockSpec((E // eb, DE), lambda h, i: (i, 0)),
            pl.BlockSpec((DE, C), lambda h, i: (0, h)),
        ],
        out_specs=pl.BlockSpec((1, E // eb, C), lambda h, i: (h, i, 0)),
        out_shape=jax.ShapeDtypeStruct((H, E, C), jnp.float32),
    )(edge_attr, W_e)

    xl2 = xl.reshape(H * N, C)
    xr2 = xr.reshape(H * N, C)
    ef2 = ef.reshape(H * E, C)
    att_flat = att.reshape(H * C)

    msg_out, deg_out, den_out, _p = _sc_call(src, dst, edge_attr, xl2, xr2,
                                             ef2, att_flat)

    bias_h = bias.reshape(H, C)
    wp_h = W_pred.reshape(H, C, OUT)
    out = pl.pallas_call(
        _epi_kernel,
        grid=(nb,),
        in_specs=[
            pl.BlockSpec((H, NB, C), lambda i: (0, i, 0)),
            pl.BlockSpec((H, NB, C), lambda i: (0, i, 0)),
            pl.BlockSpec((H, NB, C), lambda i: (0, i, 0)),
            pl.BlockSpec((NC, NB, C), lambda i: (0, i, 0)),
            pl.BlockSpec((NC, NB, C), lambda i: (0, i, 0)),
            pl.BlockSpec((DE, H * C), lambda i: (0, 0)),
            pl.BlockSpec((H, C), lambda i: (0, 0)),
            pl.BlockSpec((H, C), lambda i: (0, 0)),
            pl.BlockSpec((H, C, OUT), lambda i: (0, 0, 0)),
            pl.BlockSpec((OUT,), lambda i: (0,)),
        ],
        out_specs=pl.BlockSpec((NB, OUT), lambda i: (i, 0)),
        out_shape=jax.ShapeDtypeStruct((N, OUT), jnp.float32),
    )(msg_out, xl, xr, deg_out, den_out, W_e, att, bias_h, wp_h, b_pred)
    return out
